# Initial kernel scaffold; baseline (speedup 1.0000x reference)
import jax
import jax.numpy as jnp
from jax.experimental import pallas as pl


def kernel(x, w_gate, w_noise, bias):
    raise NotImplementedError("write your pallas kernel here")


# speedup vs baseline: 4.9730x; 4.9730x over previous
import functools

import jax
import jax.numpy as jnp
import numpy as np
from jax import lax
from jax.experimental import pallas as pl
from jax.experimental.pallas import tpu as pltpu

_INPUT_DIM = 4096
_NUM_EXPERTS = 64
_TOP_K = 8
_NOISE_EPS = 0.01
_BATCH = 8192
_BLK = 512

_STD_NORMAL = np.asarray(
    jax.random.normal(jax.random.key(42), (_BATCH, _NUM_EXPERTS),
                      dtype=jnp.float32))


def _gating_kernel(x_ref, w_ref, std_ref, bias_ref, gk_ref, router_ref,
                   load_ref):
    logits = jnp.dot(x_ref[...], w_ref[...],
                     preferred_element_type=jnp.float32)
    clean = logits[:, :_NUM_EXPERTS]
    raw_noise = logits[:, _NUM_EXPERTS:]
    noise = std_ref[...] * jax.nn.softplus(raw_noise) * _NOISE_EPS
    gates = jax.nn.sigmoid(clean + noise)
    bias_gates = gates + bias_ref[...]

    iota = lax.broadcasted_iota(jnp.int32, (_BLK, _NUM_EXPERTS), 1)
    router = jnp.zeros((_BLK, _NUM_EXPERTS), dtype=jnp.bool_)
    work = bias_gates
    cols = []
    for _ in range(_TOP_K):
        m = jnp.max(work, axis=1, keepdims=True)
        cand = jnp.where(work == m, iota, _NUM_EXPERTS)
        idx = jnp.min(cand, axis=1, keepdims=True)
        sel = iota == idx
        cols.append(jnp.sum(jnp.where(sel, gates, 0.0), axis=1,
                            keepdims=True))
        work = jnp.where(sel, -jnp.inf, work)
        router = jnp.logical_or(router, sel)

    gk_ref[...] = jnp.concatenate(cols, axis=1)
    router_i = router.astype(jnp.int32)
    router_ref[...] = router_i

    part = jnp.sum(router_i.astype(jnp.float32), axis=0, keepdims=True)

    @pl.when(pl.program_id(0) == 0)
    def _init():
        load_ref[...] = jnp.zeros_like(load_ref)

    load_ref[...] += part * (1.0 / (_BATCH * _TOP_K))


@jax.jit
def _gating(x, w_comb, std, bias_row):
    gk, router, load = pl.pallas_call(
        _gating_kernel,
        grid=(_BATCH // _BLK,),
        in_specs=[
            pl.BlockSpec((_BLK, _INPUT_DIM), lambda i: (i, 0)),
            pl.BlockSpec((_INPUT_DIM, 2 * _NUM_EXPERTS), lambda i: (0, 0)),
            pl.BlockSpec((_BLK, _NUM_EXPERTS), lambda i: (i, 0)),
            pl.BlockSpec((1, _NUM_EXPERTS), lambda i: (0, 0)),
        ],
        out_specs=[
            pl.BlockSpec((_BLK, _TOP_K), lambda i: (i, 0)),
            pl.BlockSpec((_BLK, _NUM_EXPERTS), lambda i: (i, 0)),
            pl.BlockSpec((1, _NUM_EXPERTS), lambda i: (0, 0)),
        ],
        out_shape=[
            jax.ShapeDtypeStruct((_BATCH, _TOP_K), jnp.float32),
            jax.ShapeDtypeStruct((_BATCH, _NUM_EXPERTS), jnp.int32),
            jax.ShapeDtypeStruct((1, _NUM_EXPERTS), jnp.float32),
        ],
        compiler_params=pltpu.CompilerParams(
            dimension_semantics=("arbitrary",)),
    )(x, w_comb, std, bias_row)
    return gk, router, load.reshape(_NUM_EXPERTS)


def kernel(x, w_gate, w_noise, bias):
    w_comb = jnp.concatenate([w_gate, w_noise], axis=0).T
    std = jnp.asarray(_STD_NORMAL)
    return _gating(x, w_comb, std, bias.reshape(1, _NUM_EXPERTS))
